# baseline (speedup 1.0000x reference)
import functools

import jax
import jax.numpy as jnp
from jax.experimental import pallas as pl
from jax.experimental.pallas import tpu as pltpu

BN_EPS = 1e-5
KSIZE = 3
PAD = 1
VMEM_LIMIT_BYTES = 100 * 1024 * 1024


def _round_up(x, m):
    return (x + m - 1) // m * m


def _stats(acc, mask, sum_ref, ssq_ref):
    yv = acc * mask
    s = jnp.sum(yv, axis=0, keepdims=True)
    q = jnp.sum(yv * acc, axis=0, keepdims=True)
    sum_ref[...] = jnp.broadcast_to(s[None], sum_ref.shape)
    ssq_ref[...] = jnp.broadcast_to(q[None], ssq_ref.shape)


def _conv_stats_kernel(x_ref, w_ref, mask_ref, y_ref, sum_ref, ssq_ref,
                       im_ref, *, m_rows, cin, tap_offsets):
    for t, off in enumerate(tap_offsets):
        im_ref[:, pl.ds(t * cin, cin)] = x_ref[0, pl.ds(off, m_rows), :]
    acc = jnp.dot(im_ref[...], w_ref[...], preferred_element_type=jnp.float32)
    _stats(acc, mask_ref[...], sum_ref, ssq_ref)
    y_ref[0] = acc.astype(y_ref.dtype)


def _bn_conv_stats_kernel(y1_ref, a_ref, c_ref, w_ref, mask_ref,
                          y2_ref, sum_ref, ssq_ref, im_ref, *,
                          m_rows, cin, w_pad, tap_offsets):
    z = jnp.maximum(y1_ref[0].astype(jnp.float32) * a_ref[...] + c_ref[...],
                    0.0) * mask_ref[...]
    zb = z.astype(jnp.bfloat16)
    cz = jnp.zeros((abs(tap_offsets[-1] - (w_pad + 1)) + 1, zb.shape[1]),
                   jnp.bfloat16)
    for t, off in enumerate(tap_offsets):
        sh = off - (w_pad + 1)
        if sh >= 0:
            im_ref[pl.ds(0, m_rows - sh), pl.ds(t * cin, cin)] = zb[sh:]
            if sh:
                im_ref[pl.ds(m_rows - sh, sh), pl.ds(t * cin, cin)] = cz[:sh]
        else:
            im_ref[pl.ds(-sh, m_rows + sh), pl.ds(t * cin, cin)] = zb[:sh]
            im_ref[pl.ds(0, -sh), pl.ds(t * cin, cin)] = cz[:-sh]
    acc = jnp.dot(im_ref[...], w_ref[...], preferred_element_type=jnp.float32)
    _stats(acc, mask_ref[...], sum_ref, ssq_ref)
    y2_ref[0] = acc.astype(y2_ref.dtype)


def _bn_relu_kernel(y_ref, a_ref, c_ref, o_ref):
    o_ref[0] = jnp.maximum(
        y_ref[0].astype(jnp.float32) * a_ref[...] + c_ref[...],
        0.0).astype(o_ref.dtype)


def _fold_bn(sums, ssqs, gamma, beta, count, cout):
    ch_sum = jnp.sum(sums[:, 0, :], axis=0)
    ch_ssq = jnp.sum(ssqs[:, 0, :], axis=0)
    mean = ch_sum / count
    var = jnp.maximum(ch_ssq / count - mean * mean, 0.0)
    scale = gamma * jax.lax.rsqrt(var + BN_EPS)
    a = scale.reshape(1, cout)
    c = (beta - mean * scale).reshape(1, cout)
    return a, c


def _weight_im2col(weight):
    w = jnp.transpose(weight, (2, 3, 1, 0))
    k = weight.shape[-1]
    return w.reshape(k * k * weight.shape[1], weight.shape[0]).astype(
        jnp.bfloat16)


def kernel(x, l1_w, l1_b, l1_g, l1_beta, l2_w, l2_b, l2_g, l2_beta):
    del l1_b, l2_b
    n, cin, h, w = x.shape
    mid = l1_w.shape[0]
    cout = l2_w.shape[0]
    h_pad, w_pad = h + 2 * PAD, w + 2 * PAD
    h_out, w_out = h_pad - KSIZE + 1, w_pad - KSIZE + 1
    m_rows = h_out * w_pad
    p_in = _round_up(h_pad * w_pad + KSIZE - 1, 16)
    tap_offsets = tuple(kh * w_pad + kw
                        for kh in range(KSIZE) for kw in range(KSIZE))
    count = n * h_out * w_out

    xt = jnp.transpose(x, (0, 2, 3, 1))
    xp = jnp.pad(xt, ((0, 0), (PAD, PAD), (PAD, PAD), (0, 0)))
    x_flat = xp.reshape(n, h_pad * w_pad, cin)
    x_flat = jnp.pad(x_flat, ((0, 0), (0, p_in - h_pad * w_pad), (0, 0)))
    x_flat = x_flat.astype(jnp.bfloat16)
    w1 = _weight_im2col(l1_w)
    w2 = _weight_im2col(l2_w)

    col = jax.lax.broadcasted_iota(jnp.int32, (m_rows, 1), 0) % w_pad
    mask = jnp.broadcast_to((col < w_out).astype(jnp.float32), (m_rows, mid))

    conv_flops = 2 * n * m_rows * KSIZE * KSIZE * cin * mid
    grid = (n,)
    stats_specs = [
        pl.BlockSpec((1, 8, mid), lambda i: (i, 0, 0)),
        pl.BlockSpec((1, 8, mid), lambda i: (i, 0, 0)),
    ]
    mask_spec = pl.BlockSpec((m_rows, mid), lambda i: (0, 0))
    kk_cin = KSIZE * KSIZE * cin

    y1, s1, q1 = pl.pallas_call(
        functools.partial(_conv_stats_kernel, m_rows=m_rows, cin=cin,
                          tap_offsets=tap_offsets),
        out_shape=(
            jax.ShapeDtypeStruct((n, m_rows, mid), jnp.bfloat16),
            jax.ShapeDtypeStruct((n, 8, mid), jnp.float32),
            jax.ShapeDtypeStruct((n, 8, mid), jnp.float32),
        ),
        grid_spec=pltpu.PrefetchScalarGridSpec(
            num_scalar_prefetch=0,
            grid=grid,
            in_specs=[
                pl.BlockSpec((1, p_in, cin), lambda i: (i, 0, 0)),
                pl.BlockSpec((kk_cin, mid), lambda i: (0, 0)),
                mask_spec,
            ],
            out_specs=[pl.BlockSpec((1, m_rows, mid), lambda i: (i, 0, 0))]
            + stats_specs,
            scratch_shapes=[pltpu.VMEM((m_rows, kk_cin), jnp.bfloat16)],
        ),
        compiler_params=pltpu.CompilerParams(
            dimension_semantics=("parallel",),
            vmem_limit_bytes=VMEM_LIMIT_BYTES,
        ),
        cost_estimate=pl.CostEstimate(
            flops=conv_flops, transcendentals=0,
            bytes_accessed=2 * (n * p_in * cin + n * m_rows * mid)),
    )(x_flat, w1, mask)

    a1, c1 = _fold_bn(s1, q1, l1_g, l1_beta, count, mid)

    y2, s2, q2 = pl.pallas_call(
        functools.partial(_bn_conv_stats_kernel, m_rows=m_rows, cin=mid,
                          w_pad=w_pad, tap_offsets=tap_offsets),
        out_shape=(
            jax.ShapeDtypeStruct((n, m_rows, cout), jnp.bfloat16),
            jax.ShapeDtypeStruct((n, 8, cout), jnp.float32),
            jax.ShapeDtypeStruct((n, 8, cout), jnp.float32),
        ),
        grid_spec=pltpu.PrefetchScalarGridSpec(
            num_scalar_prefetch=0,
            grid=grid,
            in_specs=[
                pl.BlockSpec((1, m_rows, mid), lambda i: (i, 0, 0)),
                pl.BlockSpec((1, mid), lambda i: (0, 0)),
                pl.BlockSpec((1, mid), lambda i: (0, 0)),
                pl.BlockSpec((KSIZE * KSIZE * mid, cout), lambda i: (0, 0)),
                mask_spec,
            ],
            out_specs=[pl.BlockSpec((1, m_rows, cout), lambda i: (i, 0, 0))]
            + stats_specs,
            scratch_shapes=[pltpu.VMEM((m_rows, KSIZE * KSIZE * mid),
                                       jnp.bfloat16)],
        ),
        compiler_params=pltpu.CompilerParams(
            dimension_semantics=("parallel",),
            vmem_limit_bytes=VMEM_LIMIT_BYTES,
        ),
        cost_estimate=pl.CostEstimate(
            flops=conv_flops, transcendentals=0,
            bytes_accessed=2 * (n * m_rows * mid + n * m_rows * cout)),
    )(y1, a1, c1, w2, mask)

    a2, c2 = _fold_bn(s2, q2, l2_g, l2_beta, count, cout)

    out_flat = pl.pallas_call(
        _bn_relu_kernel,
        out_shape=jax.ShapeDtypeStruct((n, m_rows, cout), jnp.bfloat16),
        grid_spec=pltpu.PrefetchScalarGridSpec(
            num_scalar_prefetch=0,
            grid=grid,
            in_specs=[
                pl.BlockSpec((1, m_rows, cout), lambda i: (i, 0, 0)),
                pl.BlockSpec((1, cout), lambda i: (0, 0)),
                pl.BlockSpec((1, cout), lambda i: (0, 0)),
            ],
            out_specs=pl.BlockSpec((1, m_rows, cout), lambda i: (i, 0, 0)),
        ),
        compiler_params=pltpu.CompilerParams(
            dimension_semantics=("parallel",),
            vmem_limit_bytes=VMEM_LIMIT_BYTES,
        ),
        cost_estimate=pl.CostEstimate(
            flops=2 * n * m_rows * cout, transcendentals=0,
            bytes_accessed=4 * n * m_rows * cout),
    )(y2, a2, c2)

    out = out_flat.reshape(n, h_out, w_pad, cout)[:, :, :w_out, :]
    return jnp.transpose(out, (0, 3, 1, 2)).astype(jnp.float32)
